# baseline (speedup 1.0000x reference)
import dataclasses
import functools

import jax
import jax.numpy as jnp
from jax import lax
from jax.experimental import pallas as pl
from jax.experimental.pallas import tpu as pltpu
from jax.experimental.pallas import tpu_sc as plsc

_NC, _NS, _L = 2, 16, 16
_NW = _NC * _NS
_K = 128
_ZR = 64
_SPMEM_WORDS = 2_020_000


def _sc_mesh():
    return plsc.VectorSubcoreMesh(core_axis_name="c", subcore_axis_name="s")


def _sc_params(layout_passes=True):
    cp = pltpu.CompilerParams(use_tc_tiling_on_sc=False)
    if not layout_passes and (
            "needs_layout_passes" in pltpu.CompilerParams.__dataclass_fields__):
        cp = dataclasses.replace(cp, needs_layout_passes=False)
    return cp


@functools.cache
def _segsum_kernel(c, nd_pad, e_pad, nb=2):
    epw = e_pad // _NW
    nch = epw // _K
    rps = nd_pad // _NS

    @functools.partial(
        pl.kernel,
        out_type=jax.ShapeDtypeStruct((_NC, nd_pad, c), jnp.float32),
        mesh=_sc_mesh(),
        scratch_types=[
            pltpu.VMEM((nch, _K), jnp.int32),
            pltpu.VMEM((nch, _K), jnp.int32),
        ] + [pltpu.VMEM((_K, c), jnp.float32)] * nb + [
            pltpu.VMEM((_ZR, c), jnp.float32),
            pltpu.VMEM_SHARED((nd_pad, c), jnp.float32),
        ] + [pltpu.SemaphoreType.DMA] * (2 * nb),
        compiler_params=_sc_params(),
    )
    def k(x_hbm, row_hbm, col_hbm, out_hbm, row_v, col_v, *rest):
        bufs = rest[:nb]
        ztile = rest[nb]
        acc = rest[nb + 1]
        gsems = rest[nb + 2:2 * nb + 2]
        ssems = rest[2 * nb + 2:]
        cid = lax.axis_index("c")
        sid = lax.axis_index("s")
        zv = jnp.zeros((_L,), jnp.float32)

        @pl.loop(0, _ZR)
        def _(i):
            @pl.loop(0, c, step=_L)
            def _(j):
                ztile[i, pl.ds(j, _L)] = zv

        rbase = sid * rps

        @pl.loop(0, rps, step=_ZR)
        def _(r):
            pltpu.sync_copy(ztile, acc.at[pl.ds(rbase + r, _ZR)])

        wid = sid * _NC + cid
        pltpu.sync_copy(row_hbm.at[pl.ds(wid * nch, nch)], row_v)
        pltpu.sync_copy(col_hbm.at[pl.ds(wid * nch, nch)], col_v)
        plsc.subcore_barrier()

        hg = [None] * nch
        hs = [None] * nch
        for i in range(min(nb, nch)):
            hg[i] = pltpu.async_copy(x_hbm.at[row_v.at[i]], bufs[i],
                                     gsems[i])
        for i in range(nch):
            b = i % nb
            hg[i].wait()
            if i >= 1:
                hs[i - 1].wait()
                f = i + nb - 1
                if f < nch:
                    fb = f % nb
                    hg[f] = pltpu.async_copy(x_hbm.at[row_v.at[f]],
                                             bufs[fb], gsems[fb])
            hs[i] = pltpu.async_copy(bufs[b], acc.at[col_v.at[i]],
                                     ssems[b], add=True)
        hs[nch - 1].wait()
        plsc.subcore_barrier()
        pltpu.sync_copy(acc.at[pl.ds(rbase, rps)],
                        out_hbm.at[cid].at[pl.ds(rbase, rps)])

    return k


def _pad1(a, n, fill):
    if n == a.shape[0]:
        return a
    return jnp.concatenate(
        [a, jnp.full((n - a.shape[0],), fill, a.dtype)])


def _ceil_to(x, m):
    return -(-x // m) * m


def _sc_segsum_parts(x, rowp, colp, nd_pad):
    ns, c = x.shape
    e_pad = rowp.shape[0] * rowp.shape[1]
    nch = e_pad // _NW // _K

    def words(cw, nb):
        return _NS * (2 * nch * _K + nb * _K * cw + _ZR * cw) + nd_pad * cw

    cw = c
    while words(cw, 2) > _SPMEM_WORDS:
        cw //= 2
    nb = 2
    for cand in (4, 3):
        if words(cw, cand) <= _SPMEM_WORDS:
            nb = cand
            break
    return [_segsum_kernel(cw, nd_pad, e_pad, nb)(x[:, i:i + cw], rowp, colp)
            for i in range(0, c, cw)]


@functools.cache
def _count_kernel(nd_pad, e_pad):
    epw = e_pad // _NW
    nchunks = epw // _K
    rps = nd_pad // _NS

    @functools.partial(
        pl.kernel,
        out_type=jax.ShapeDtypeStruct((_NC, nd_pad, 16), jnp.float32),
        mesh=_sc_mesh(),
        scratch_types=[
            pltpu.VMEM((nchunks, _K), jnp.int32),
            pltpu.VMEM((_K, 16), jnp.float32),
            pltpu.VMEM((_ZR, 16), jnp.float32),
            pltpu.VMEM_SHARED((nd_pad, 16), jnp.float32),
            pltpu.SemaphoreType.DMA,
        ],
        compiler_params=pltpu.CompilerParams(use_tc_tiling_on_sc=False),
    )
    def k(col_hbm, out_hbm, col_v, ones_v, ztile, acc, sem):
        cid = lax.axis_index("c")
        sid = lax.axis_index("s")
        zv = jnp.zeros((_L,), jnp.float32)
        ov = jnp.ones((_L,), jnp.float32)

        @pl.loop(0, _ZR)
        def _(i):
            ztile[i, pl.ds(0, _L)] = zv

        @pl.loop(0, _K)
        def _(i):
            ones_v[i, pl.ds(0, _L)] = ov

        rbase = sid * rps

        @pl.loop(0, rps, step=_ZR)
        def _(r):
            pltpu.sync_copy(ztile, acc.at[pl.ds(rbase + r, _ZR)])

        wid = sid * _NC + cid
        pltpu.sync_copy(col_hbm.at[pl.ds(wid * nchunks, nchunks)], col_v)
        plsc.subcore_barrier()

        hs = [pltpu.async_copy(ones_v, acc.at[col_v.at[i]], sem, add=True)
              for i in range(nchunks)]
        for h in hs:
            h.wait()

        plsc.subcore_barrier()
        pltpu.sync_copy(acc.at[pl.ds(rbase, rps)],
                        out_hbm.at[cid].at[pl.ds(rbase, rps)])

    return k


@functools.cache
def _winner_kernel(nd_pad, e_pad):
    epw = e_pad // _NW
    nchunks = epw // _K

    @functools.partial(
        pl.kernel,
        out_type=jax.ShapeDtypeStruct((_NW, nd_pad), jnp.int32),
        mesh=_sc_mesh(),
        scratch_types=[
            pltpu.VMEM((epw,), jnp.int32),
            pltpu.VMEM((nd_pad,), jnp.int32),
        ],
        compiler_params=_sc_params(layout_passes=False),
    )
    def k(col_hbm, out_hbm, col_v, wtab):
        cid = lax.axis_index("c")
        sid = lax.axis_index("s")
        wid = sid * _NC + cid
        neg = jnp.full((_L,), -1, jnp.int32)
        pltpu.sync_copy(col_hbm.at[pl.ds(wid * epw, epw)], col_v)

        @pl.loop(0, nd_pad, step=_L)
        def _(i):
            wtab[pl.ds(i, _L)] = neg

        iota = lax.iota(jnp.int32, _L)
        nxt_idx = jnp.minimum(iota + 1, _L - 1)
        base = wid * epw

        @pl.loop(0, epw, step=_L)
        def _(j):
            if True:
                c = col_v[pl.ds(j, _L)]
                eid = base + j + iota
                key = (c << 4) | iota
                sk, se = plsc.sort_key_val(key, eid)
                cs = sk >> 4
                nxt = lax.gather(
                    cs, nxt_idx[:, None],
                    lax.GatherDimensionNumbers(
                        offset_dims=(), collapsed_slice_dims=(0,),
                        start_index_map=(0,)),
                    slice_sizes=(1,),
                    mode=lax.GatherScatterMode.PROMISE_IN_BOUNDS)
                lastm = (cs != nxt) | (iota == _L - 1)
                plsc.store_scatter(wtab, [cs], se, mask=lastm)

        pltpu.sync_copy(wtab, out_hbm.at[wid])

    return k


@functools.cache
def _gather16_kernel(ne, n_idx):
    ipw = n_idx // _NW
    nchunks = ipw // _K

    @functools.partial(
        pl.kernel,
        out_type=jax.ShapeDtypeStruct((n_idx, 16), jnp.float32),
        mesh=_sc_mesh(),
        scratch_types=[
            pltpu.VMEM((nchunks, _K), jnp.int32),
            pltpu.VMEM((_K, 16), jnp.float32),
            pltpu.VMEM((_K, 16), jnp.float32),
            pltpu.SemaphoreType.DMA,
            pltpu.SemaphoreType.DMA,
            pltpu.SemaphoreType.DMA,
            pltpu.SemaphoreType.DMA,
        ],
        compiler_params=pltpu.CompilerParams(use_tc_tiling_on_sc=False),
    )
    def k(tab_hbm, idx_hbm, out_hbm, idx_v, ga, gb, gsa, gsb, osa, osb):
        cid = lax.axis_index("c")
        sid = lax.axis_index("s")
        wid = sid * _NC + cid
        base = wid * ipw
        pltpu.sync_copy(idx_hbm.at[pl.ds(wid * nchunks, nchunks)], idx_v)

        bufs = (ga, gb)
        gsems = (gsa, gsb)
        osems = (osa, osb)
        hg = [None] * nchunks
        ho = [None] * nchunks
        hg[0] = pltpu.async_copy(tab_hbm.at[idx_v.at[0]], ga, gsa)
        for i in range(nchunks):
            b = i % 2
            hg[i].wait()
            if i >= 1:
                ho[i - 1].wait()
            if i + 1 < nchunks:
                nb = (i + 1) % 2
                hg[i + 1] = pltpu.async_copy(
                    tab_hbm.at[idx_v.at[i + 1]], bufs[nb], gsems[nb])
            ho[i] = pltpu.async_copy(
                bufs[b], out_hbm.at[pl.ds(base + i * _K, _K)], osems[b])
        ho[nchunks - 1].wait()

    return k


def _tc_winner_combine(wtabs, emax):
    nw, ndp = wtabs.shape
    tmc = 1024

    def body(w_ref, o_ref):
        o_ref[...] = jnp.clip(jnp.max(w_ref[...], axis=0), 0, emax)

    return pl.pallas_call(
        body,
        grid=(ndp // tmc,),
        in_specs=[pl.BlockSpec((nw, tmc), lambda i: (0, i))],
        out_specs=pl.BlockSpec((tmc,), lambda i: (i,)),
        out_shape=jax.ShapeDtypeStruct((ndp,), jnp.int32),
    )(wtabs)

_TM = 512


def _tc_linear_stats(x, w, b, n_valid):
    npad, cin = x.shape
    cout = w.shape[1]
    grid = npad // _TM

    def body(x_ref, w_ref, b_ref, h_ref, st_ref, acc):
        i = pl.program_id(0)
        h = jnp.dot(x_ref[...], w_ref[...],
                    preferred_element_type=jnp.float32) + b_ref[...]
        h_ref[...] = h
        rows = jax.lax.broadcasted_iota(jnp.int32, (_TM, 1), 0) + i * _TM
        hm = jnp.where(rows < n_valid, h, 0.0)

        @pl.when(i == 0)
        def _():
            acc[...] = jnp.zeros_like(acc)

        acc[0, :] += jnp.sum(hm, 0)
        acc[1, :] += jnp.sum(hm * hm, 0)

        @pl.when(i == grid - 1)
        def _():
            st_ref[...] = acc[...]

    return pl.pallas_call(
        body,
        grid=(grid,),
        in_specs=[
            pl.BlockSpec((_TM, cin), lambda i: (i, 0)),
            pl.BlockSpec((cin, cout), lambda i: (0, 0)),
            pl.BlockSpec((cout,), lambda i: (0,)),
        ],
        out_specs=[
            pl.BlockSpec((_TM, cout), lambda i: (i, 0)),
            pl.BlockSpec((2, cout), lambda i: (0, 0)),
        ],
        out_shape=[
            jax.ShapeDtypeStruct((npad, cout), jnp.float32),
            jax.ShapeDtypeStruct((2, cout), jnp.float32),
        ],
        scratch_shapes=[pltpu.VMEM((2, cout), jnp.float32)],
    )(x, w, b)


def _tc_scale_shift_act(h, scale, shift):
    npad, c = h.shape

    def body(h_ref, sc_ref, sh_ref, o_ref):
        o_ref[...] = jnp.maximum(h_ref[...] * sc_ref[...] + sh_ref[...], 0.0)

    return pl.pallas_call(
        body,
        grid=(npad // _TM,),
        in_specs=[
            pl.BlockSpec((_TM, c), lambda i: (i, 0)),
            pl.BlockSpec((c,), lambda i: (0,)),
            pl.BlockSpec((c,), lambda i: (0,)),
        ],
        out_specs=pl.BlockSpec((_TM, c), lambda i: (i, 0)),
        out_shape=jax.ShapeDtypeStruct((npad, c), jnp.float32),
    )(h, scale, shift)


def _tc_edge_gram(ea):
    e, c = ea.shape
    tm = 2048
    epad = _ceil_to(e, tm)
    if epad != e:
        ea = jnp.concatenate([ea, jnp.zeros((epad - e, c), ea.dtype)])
    grid = epad // tm

    def body(a_ref, g_ref, s_ref, acc):
        i = pl.program_id(0)
        a = a_ref[...]

        @pl.when(i == 0)
        def _():
            acc[...] = jnp.zeros_like(acc)

        acc[:c, :] += jax.lax.dot_general(
            a, a, (((0,), (0,)), ((), ())),
            preferred_element_type=jnp.float32)
        acc[c, :] += jnp.sum(a, 0)

        @pl.when(i == grid - 1)
        def _():
            g_ref[...] = acc[:c, :]
            s_ref[...] = acc[c, :]

    return pl.pallas_call(
        body,
        grid=(grid,),
        in_specs=[pl.BlockSpec((tm, c), lambda i: (i, 0))],
        out_specs=[
            pl.BlockSpec((c, c), lambda i: (0, 0)),
            pl.BlockSpec((c,), lambda i: (0,)),
        ],
        out_shape=[
            jax.ShapeDtypeStruct((c, c), jnp.float32),
            jax.ShapeDtypeStruct((c,), jnp.float32),
        ],
        scratch_shapes=[pltpu.VMEM((c + 1, c), jnp.float32)],
    )(ea)


def _tc_relation_fwd(parts, cnt16, xd, eadw, wl, bl, wr, etw, etb,
                     attw, attb, n_valid):
    ndp = xd.shape[0]
    c = xd.shape[1]
    cw = parts[0].shape[-1]
    nparts = len(parts)
    grid = ndp // _TM
    out_dim = wl.shape[1]

    def body(*refs):
        part_refs = refs[:nparts]
        (cnt_ref, xd_ref, eadw_ref, wl_ref, bl_ref, wr_ref, etw_ref,
         etb_ref, attw_ref, attb_ref, out_ref, st_ref, acc) = refs[nparts:]
        i = pl.program_id(0)
        if nparts > 1:
            s = jnp.concatenate([p[0] + p[1] for p in part_refs], axis=-1)
        else:
            s = part_refs[0][0] + part_refs[0][1]
        cnt = cnt_ref[0, :, 0:1] + cnt_ref[1, :, 0:1]
        mean = s / jnp.maximum(cnt, 1.0)
        out = (jnp.dot(mean, wl_ref[...], preferred_element_type=jnp.float32)
               + bl_ref[...]
               + jnp.dot(xd_ref[...], wr_ref[...],
                         preferred_element_type=jnp.float32))
        eat = jnp.dot(eadw_ref[...], etw_ref[...],
                      preferred_element_type=jnp.float32) + etb_ref[...]
        score = (jnp.dot(out, attw_ref[...][:out_dim, :],
                         preferred_element_type=jnp.float32)
                 + jnp.dot(eat, attw_ref[...][out_dim:, :],
                           preferred_element_type=jnp.float32)
                 + attb_ref[0])
        attn = jax.nn.sigmoid(score)
        out = out + jnp.where(cnt > 0.0, attn * eat, 0.0)
        out_ref[...] = out
        rows = jax.lax.broadcasted_iota(jnp.int32, (_TM, 1), 0) + i * _TM
        om = jnp.where(rows < n_valid, out, 0.0)

        @pl.when(i == 0)
        def _():
            acc[...] = jnp.zeros_like(acc)

        acc[0, :] += jnp.sum(om, 0)
        acc[1, :] += jnp.sum(om * om, 0)

        @pl.when(i == grid - 1)
        def _():
            st_ref[...] = acc[...]

    in_specs = (
        [pl.BlockSpec((2, _TM, cw), lambda i: (0, i, 0))] * nparts + [
            pl.BlockSpec((2, _TM, 16), lambda i: (0, i, 0)),
            pl.BlockSpec((_TM, c), lambda i: (i, 0)),
            pl.BlockSpec((_TM, 32), lambda i: (i, 0)),
            pl.BlockSpec((c, out_dim), lambda i: (0, 0)),
            pl.BlockSpec((out_dim,), lambda i: (0,)),
            pl.BlockSpec((c, out_dim), lambda i: (0, 0)),
            pl.BlockSpec((32, out_dim), lambda i: (0, 0)),
            pl.BlockSpec((out_dim,), lambda i: (0,)),
            pl.BlockSpec((2 * out_dim, 1), lambda i: (0, 0)),
            pl.BlockSpec((1,), lambda i: (0,)),
        ])
    return pl.pallas_call(
        body,
        grid=(grid,),
        in_specs=in_specs,
        out_specs=[
            pl.BlockSpec((_TM, out_dim), lambda i: (i, 0)),
            pl.BlockSpec((2, out_dim), lambda i: (0, 0)),
        ],
        out_shape=[
            jax.ShapeDtypeStruct((ndp, out_dim), jnp.float32),
            jax.ShapeDtypeStruct((2, out_dim), jnp.float32),
        ],
        scratch_shapes=[pltpu.VMEM((2, out_dim), jnp.float32)],
    )(*parts, cnt16, xd, eadw, wl, bl, wr, etw, etb, attw, attb)


def _tc_finalize(outs, scales, shifts, x_t, wself, bself, head=None):
    ndp, c = x_t.shape
    nrel = len(outs)
    out_dim = wself.shape[1]
    grid = ndp // _TM

    def body(*refs):
        o_refs = refs[:nrel]
        sc_refs = refs[nrel:2 * nrel]
        sh_refs = refs[2 * nrel:3 * nrel]
        rest = refs[3 * nrel:]
        if head is None:
            x_ref, ws_ref, bs_ref, out_ref = rest
        else:
            x_ref, ws_ref, bs_ref, hw_ref, hb_ref, ha_ref, out_ref = rest
        acc = (jnp.dot(x_ref[...], ws_ref[...],
                       preferred_element_type=jnp.float32) + bs_ref[...])
        for r in range(nrel):
            acc = acc + jnp.maximum(
                o_refs[r][...] * sc_refs[r][...] + sh_refs[r][...], 0.0)
        xn = jnp.maximum(acc, 0.0)
        if head is None:
            out_ref[...] = xn
        else:
            y = jnp.sum(xn * hw_ref[...], axis=1) + hb_ref[0]
            out_ref[...] = jnp.where(y >= 0.0, y, ha_ref[0] * y)

    in_specs = ([pl.BlockSpec((_TM, out_dim), lambda i: (i, 0))] * nrel
                + [pl.BlockSpec((out_dim,), lambda i: (0,))] * (2 * nrel)
                + [pl.BlockSpec((_TM, c), lambda i: (i, 0)),
                   pl.BlockSpec((c, out_dim), lambda i: (0, 0)),
                   pl.BlockSpec((out_dim,), lambda i: (0,))])
    args = list(outs) + list(scales) + list(shifts) + [x_t, wself, bself]
    if head is None:
        out_spec = pl.BlockSpec((_TM, out_dim), lambda i: (i, 0))
        out_shape = jax.ShapeDtypeStruct((ndp, out_dim), jnp.float32)
    else:
        hw, hb, ha = head
        in_specs += [pl.BlockSpec((out_dim,), lambda i: (0,)),
                     pl.BlockSpec((1,), lambda i: (0,)),
                     pl.BlockSpec((1,), lambda i: (0,))]
        args += [hw, hb, ha]
        out_spec = pl.BlockSpec((_TM,), lambda i: (i,))
        out_shape = jax.ShapeDtypeStruct((ndp,), jnp.float32)
    return pl.pallas_call(
        body,
        grid=(grid,),
        in_specs=in_specs,
        out_specs=out_spec,
        out_shape=out_shape,
    )(*args)


def _bn_scale_shift(stats, n, g, be, eps=1e-5):
    mu = stats[0] / n
    var = stats[1] / n - mu * mu
    scale = g / jnp.sqrt(var + eps)
    return scale, be - mu * scale

_NODE_TYPES = ('pfas_sites', 'sw_stations', 'gw_wells')
_EDGE_TYPES = (
    ('pfas_sites', 'gw_wells'),
    ('pfas_sites', 'sw_stations'),
    ('sw_stations', 'pfas_sites'),
    ('sw_stations', 'gw_wells'),
    ('gw_wells', 'sw_stations'),
    ('gw_wells', 'gw_wells'),
    ('gw_wells', 'pfas_sites'),
)


def _ek(e):
    return e[0] + '->' + e[1]


def _pad_rows(a, n):
    if a.shape[0] == n:
        return a
    return jnp.concatenate(
        [a, jnp.zeros((n - a.shape[0],) + a.shape[1:], a.dtype)])


def kernel(x_dict, edge_index, edge_attr, params):
    n_nodes = {t: x_dict[t].shape[0] for t in _NODE_TYPES}
    ndp = {t: _ceil_to(n_nodes[t] + 1, _ZR * _NS) for t in _NODE_TYPES}

    xd = {}
    for t in _NODE_TYPES:
        q = params['node_red'][t]
        xp = _pad_rows(x_dict[t], ndp[t])
        h, st = _tc_linear_stats(xp, q['w'], q['b'], n_nodes[t])
        sc, sh = _bn_scale_shift(st, n_nodes[t], q['g'], q['be'])
        xd[t] = _tc_scale_shift_act(h, sc, sh)

    cnt16 = {}
    ead_win = {}
    rowp = {}
    colp = {}
    for e in _EDGE_TYPES:
        k = _ek(e)
        ei = edge_index[k]
        nd = n_nodes[e[1]]
        ndpd = ndp[e[1]]
        E = ei.shape[1]
        e_pad = _ceil_to(E, _NW * _K)
        rowp[k] = _pad1(ei[0], e_pad, 0).reshape(e_pad // _K, _K)
        colp[k] = _pad1(ei[1], e_pad, ndpd - 1).reshape(e_pad // _K, _K)
        cnt16[k] = _count_kernel(ndpd, e_pad)(colp[k])
        wtabs = _winner_kernel(ndpd, e_pad)(colp[k].reshape(-1))
        n_idx = _ceil_to(ndpd, _NW * _K)
        wsafe = _pad1(_tc_winner_combine(wtabs, E - 1), n_idx, 0)
        ea = edge_attr[k]
        ea_w = _gather16_kernel(E, n_idx)(
            ea, wsafe.reshape(n_idx // _K, _K))[:ndpd]
        q = params['edge_red'][k]
        G, su = _tc_edge_gram(ea)
        mu_ea = su / E
        cov = G / E - mu_ea[:, None] * mu_ea[None, :]
        mean_h = mu_ea @ q['w'] + q['b']
        var_h = jnp.sum(q['w'] * (cov @ q['w']), 0)
        sc_e = q['g'] / jnp.sqrt(var_h + 1e-5)
        sh_e = q['be'] - mean_h * sc_e
        h_w, _ = _tc_linear_stats(ea_w, q['w'], q['b'], ndp[e[1]])
        ead_win[k] = _tc_scale_shift_act(h_w, sc_e, sh_e)

    x = xd
    heads = {}
    for layer in ('conv1', 'conv2'):
        pl_ = params[layer]
        rel_out = {t: [] for t in _NODE_TYPES}
        rel_sc = {t: [] for t in _NODE_TYPES}
        rel_sh = {t: [] for t in _NODE_TYPES}
        for e in _EDGE_TYPES:
            k = _ek(e)
            p = pl_[k]
            dst = e[1]
            nd = n_nodes[dst]
            parts = _sc_segsum_parts(x[e[0]], rowp[k], colp[k], ndp[dst])
            o, st = _tc_relation_fwd(
                parts, cnt16[k], x[dst], ead_win[k],
                p['lin_l_w'], p['lin_l_b'], p['lin_r_w'],
                p['et_w'], p['et_b'], p['att_w'], p['att_b'], nd)
            sc, sh = _bn_scale_shift(st, nd, p['bn_g'], p['bn_b'])
            rel_out[dst].append(o)
            rel_sc[dst].append(2.0 * sc)
            rel_sh[dst].append(2.0 * sh)
        xn = {}
        for t in _NODE_TYPES:
            p = pl_['self:' + t]
            head = None
            if layer == 'conv2' and t != 'pfas_sites':
                head = (params['linear']['w'][:, 0],
                        params['linear']['b'],
                        params['prelu'].reshape((1,)))
            res = _tc_finalize(rel_out[t], rel_sc[t], rel_sh[t], x[t],
                               p['lin_l_w'] + p['lin_r_w'], p['lin_l_b'],
                               head=head)
            if head is None:
                xn[t] = res
            else:
                heads[t] = res
        x = xn

    pfas = x['pfas_sites'][:n_nodes['pfas_sites']]
    sw = heads['sw_stations'][:n_nodes['sw_stations'], None]
    gw = heads['gw_wells'][:n_nodes['gw_wells'], None]
    return pfas, sw, gw


# speedup vs baseline: 1.0330x; 1.0330x over previous
import dataclasses
import functools

import jax
import jax.numpy as jnp
from jax import lax
from jax.experimental import pallas as pl
from jax.experimental.pallas import tpu as pltpu
from jax.experimental.pallas import tpu_sc as plsc

_NC, _NS, _L = 2, 16, 16
_NW = _NC * _NS
_K = 128
_ZR = 64
_SPMEM_WORDS = 2_020_000


def _sc_mesh():
    return plsc.VectorSubcoreMesh(core_axis_name="c", subcore_axis_name="s")


def _sc_params(layout_passes=True):
    cp = pltpu.CompilerParams(use_tc_tiling_on_sc=False)
    if not layout_passes and (
            "needs_layout_passes" in pltpu.CompilerParams.__dataclass_fields__):
        cp = dataclasses.replace(cp, needs_layout_passes=False)
    return cp


@functools.cache
def _segsum_kernel(c, nd_pad, e_pad, nb=2):
    epw = e_pad // _NW
    nch = epw // _K
    rps = nd_pad // _NS

    @functools.partial(
        pl.kernel,
        out_type=jax.ShapeDtypeStruct((_NC, nd_pad, c), jnp.float32),
        mesh=_sc_mesh(),
        scratch_types=[
            pltpu.VMEM((nch, _K), jnp.int32),
            pltpu.VMEM((nch, _K), jnp.int32),
        ] + [pltpu.VMEM((_K, c), jnp.float32)] * nb + [
            pltpu.VMEM((_ZR, c), jnp.float32),
            pltpu.VMEM_SHARED((nd_pad, c), jnp.float32),
        ] + [pltpu.SemaphoreType.DMA] * (2 * nb),
        compiler_params=_sc_params(),
    )
    def k(x_hbm, row_hbm, col_hbm, out_hbm, row_v, col_v, *rest):
        bufs = rest[:nb]
        ztile = rest[nb]
        acc = rest[nb + 1]
        gsems = rest[nb + 2:2 * nb + 2]
        ssems = rest[2 * nb + 2:]
        cid = lax.axis_index("c")
        sid = lax.axis_index("s")
        zv = jnp.zeros((_L,), jnp.float32)

        @pl.loop(0, _ZR)
        def _(i):
            @pl.loop(0, c, step=_L)
            def _(j):
                ztile[i, pl.ds(j, _L)] = zv

        rbase = sid * rps

        @pl.loop(0, rps, step=_ZR)
        def _(r):
            pltpu.sync_copy(ztile, acc.at[pl.ds(rbase + r, _ZR)])

        wid = sid * _NC + cid
        pltpu.sync_copy(row_hbm.at[pl.ds(wid * nch, nch)], row_v)
        pltpu.sync_copy(col_hbm.at[pl.ds(wid * nch, nch)], col_v)
        plsc.subcore_barrier()

        hg = [None] * nch
        hs = [None] * nch
        for i in range(min(nb, nch)):
            hg[i] = pltpu.async_copy(x_hbm.at[row_v.at[i]], bufs[i],
                                     gsems[i])
        for i in range(nch):
            b = i % nb
            hg[i].wait()
            if i >= 1:
                hs[i - 1].wait()
                f = i + nb - 1
                if f < nch:
                    fb = f % nb
                    hg[f] = pltpu.async_copy(x_hbm.at[row_v.at[f]],
                                             bufs[fb], gsems[fb])
            hs[i] = pltpu.async_copy(bufs[b], acc.at[col_v.at[i]],
                                     ssems[b], add=True)
        hs[nch - 1].wait()
        plsc.subcore_barrier()
        pltpu.sync_copy(acc.at[pl.ds(rbase, rps)],
                        out_hbm.at[cid].at[pl.ds(rbase, rps)])

    return k


def _pad1(a, n, fill):
    if n == a.shape[0]:
        return a
    return jnp.concatenate(
        [a, jnp.full((n - a.shape[0],), fill, a.dtype)])


def _ceil_to(x, m):
    return -(-x // m) * m


def _sc_segsum_parts(x, rowp, colp, nd_pad):
    ns, c = x.shape
    e_pad = rowp.shape[0] * rowp.shape[1]
    nch = e_pad // _NW // _K

    def words(cw, nb):
        return _NS * (2 * nch * _K + nb * _K * cw + _ZR * cw) + nd_pad * cw

    cw = c
    while words(cw, 2) > _SPMEM_WORDS:
        cw //= 2
    return [_segsum_kernel(cw, nd_pad, e_pad, 2)(x[:, i:i + cw], rowp, colp)
            for i in range(0, c, cw)]


@functools.cache
def _count_kernel(nd_pad, e_pad):
    epw = e_pad // _NW
    nchunks = epw // _K
    rps = nd_pad // _NS

    @functools.partial(
        pl.kernel,
        out_type=jax.ShapeDtypeStruct((_NC, nd_pad, 16), jnp.float32),
        mesh=_sc_mesh(),
        scratch_types=[
            pltpu.VMEM((nchunks, _K), jnp.int32),
            pltpu.VMEM((_K, 16), jnp.float32),
            pltpu.VMEM((_ZR, 16), jnp.float32),
            pltpu.VMEM_SHARED((nd_pad, 16), jnp.float32),
            pltpu.SemaphoreType.DMA,
        ],
        compiler_params=pltpu.CompilerParams(use_tc_tiling_on_sc=False),
    )
    def k(col_hbm, out_hbm, col_v, ones_v, ztile, acc, sem):
        cid = lax.axis_index("c")
        sid = lax.axis_index("s")
        zv = jnp.zeros((_L,), jnp.float32)
        ov = jnp.ones((_L,), jnp.float32)

        @pl.loop(0, _ZR)
        def _(i):
            ztile[i, pl.ds(0, _L)] = zv

        @pl.loop(0, _K)
        def _(i):
            ones_v[i, pl.ds(0, _L)] = ov

        rbase = sid * rps

        @pl.loop(0, rps, step=_ZR)
        def _(r):
            pltpu.sync_copy(ztile, acc.at[pl.ds(rbase + r, _ZR)])

        wid = sid * _NC + cid
        pltpu.sync_copy(col_hbm.at[pl.ds(wid * nchunks, nchunks)], col_v)
        plsc.subcore_barrier()

        hs = [pltpu.async_copy(ones_v, acc.at[col_v.at[i]], sem, add=True)
              for i in range(nchunks)]
        for h in hs:
            h.wait()

        plsc.subcore_barrier()
        pltpu.sync_copy(acc.at[pl.ds(rbase, rps)],
                        out_hbm.at[cid].at[pl.ds(rbase, rps)])

    return k


@functools.cache
def _winner_kernel(nd_pad, e_pad):
    epw = e_pad // _NW
    nchunks = epw // _K

    @functools.partial(
        pl.kernel,
        out_type=jax.ShapeDtypeStruct((_NW, nd_pad), jnp.int32),
        mesh=_sc_mesh(),
        scratch_types=[
            pltpu.VMEM((epw,), jnp.int32),
            pltpu.VMEM((nd_pad,), jnp.int32),
        ],
        compiler_params=_sc_params(layout_passes=False),
    )
    def k(col_hbm, out_hbm, col_v, wtab):
        cid = lax.axis_index("c")
        sid = lax.axis_index("s")
        wid = sid * _NC + cid
        neg = jnp.full((_L,), -1, jnp.int32)
        pltpu.sync_copy(col_hbm.at[pl.ds(wid * epw, epw)], col_v)

        @pl.loop(0, nd_pad, step=_L)
        def _(i):
            wtab[pl.ds(i, _L)] = neg

        iota = lax.iota(jnp.int32, _L)
        nxt_idx = jnp.minimum(iota + 1, _L - 1)
        base = wid * epw

        @pl.loop(0, epw, step=_L)
        def _(j):
            if True:
                c = col_v[pl.ds(j, _L)]
                eid = base + j + iota
                key = (c << 4) | iota
                sk, se = plsc.sort_key_val(key, eid)
                cs = sk >> 4
                nxt = lax.gather(
                    cs, nxt_idx[:, None],
                    lax.GatherDimensionNumbers(
                        offset_dims=(), collapsed_slice_dims=(0,),
                        start_index_map=(0,)),
                    slice_sizes=(1,),
                    mode=lax.GatherScatterMode.PROMISE_IN_BOUNDS)
                lastm = (cs != nxt) | (iota == _L - 1)
                plsc.store_scatter(wtab, [cs], se, mask=lastm)

        pltpu.sync_copy(wtab, out_hbm.at[wid])

    return k


@functools.cache
def _gather16_kernel(ne, n_idx):
    ipw = n_idx // _NW
    nchunks = ipw // _K

    @functools.partial(
        pl.kernel,
        out_type=jax.ShapeDtypeStruct((n_idx, 16), jnp.float32),
        mesh=_sc_mesh(),
        scratch_types=[
            pltpu.VMEM((nchunks, _K), jnp.int32),
            pltpu.VMEM((_K, 16), jnp.float32),
            pltpu.VMEM((_K, 16), jnp.float32),
            pltpu.SemaphoreType.DMA,
            pltpu.SemaphoreType.DMA,
            pltpu.SemaphoreType.DMA,
            pltpu.SemaphoreType.DMA,
        ],
        compiler_params=pltpu.CompilerParams(use_tc_tiling_on_sc=False),
    )
    def k(tab_hbm, idx_hbm, out_hbm, idx_v, ga, gb, gsa, gsb, osa, osb):
        cid = lax.axis_index("c")
        sid = lax.axis_index("s")
        wid = sid * _NC + cid
        base = wid * ipw
        pltpu.sync_copy(idx_hbm.at[pl.ds(wid * nchunks, nchunks)], idx_v)

        bufs = (ga, gb)
        gsems = (gsa, gsb)
        osems = (osa, osb)
        hg = [None] * nchunks
        ho = [None] * nchunks
        hg[0] = pltpu.async_copy(tab_hbm.at[idx_v.at[0]], ga, gsa)
        for i in range(nchunks):
            b = i % 2
            hg[i].wait()
            if i >= 1:
                ho[i - 1].wait()
            if i + 1 < nchunks:
                nb = (i + 1) % 2
                hg[i + 1] = pltpu.async_copy(
                    tab_hbm.at[idx_v.at[i + 1]], bufs[nb], gsems[nb])
            ho[i] = pltpu.async_copy(
                bufs[b], out_hbm.at[pl.ds(base + i * _K, _K)], osems[b])
        ho[nchunks - 1].wait()

    return k


def _tc_winner_combine(wtabs, emax):
    nw, ndp = wtabs.shape
    tmc = 1024

    def body(w_ref, o_ref):
        o_ref[...] = jnp.clip(jnp.max(w_ref[...], axis=0), 0, emax)

    return pl.pallas_call(
        body,
        grid=(ndp // tmc,),
        in_specs=[pl.BlockSpec((nw, tmc), lambda i: (0, i))],
        out_specs=pl.BlockSpec((tmc,), lambda i: (i,)),
        out_shape=jax.ShapeDtypeStruct((ndp,), jnp.int32),
    )(wtabs)

_TM = 512


def _tc_linear_stats(x, w, b, n_valid):
    npad, cin = x.shape
    cout = w.shape[1]
    grid = npad // _TM

    def body(x_ref, w_ref, b_ref, h_ref, st_ref, acc):
        i = pl.program_id(0)
        h = jnp.dot(x_ref[...], w_ref[...],
                    preferred_element_type=jnp.float32) + b_ref[...]
        h_ref[...] = h
        rows = jax.lax.broadcasted_iota(jnp.int32, (_TM, 1), 0) + i * _TM
        hm = jnp.where(rows < n_valid, h, 0.0)

        @pl.when(i == 0)
        def _():
            acc[...] = jnp.zeros_like(acc)

        acc[0, :] += jnp.sum(hm, 0)
        acc[1, :] += jnp.sum(hm * hm, 0)

        @pl.when(i == grid - 1)
        def _():
            st_ref[...] = acc[...]

    return pl.pallas_call(
        body,
        grid=(grid,),
        in_specs=[
            pl.BlockSpec((_TM, cin), lambda i: (i, 0)),
            pl.BlockSpec((cin, cout), lambda i: (0, 0)),
            pl.BlockSpec((cout,), lambda i: (0,)),
        ],
        out_specs=[
            pl.BlockSpec((_TM, cout), lambda i: (i, 0)),
            pl.BlockSpec((2, cout), lambda i: (0, 0)),
        ],
        out_shape=[
            jax.ShapeDtypeStruct((npad, cout), jnp.float32),
            jax.ShapeDtypeStruct((2, cout), jnp.float32),
        ],
        scratch_shapes=[pltpu.VMEM((2, cout), jnp.float32)],
    )(x, w, b)


def _tc_scale_shift_act(h, scale, shift):
    npad, c = h.shape

    def body(h_ref, sc_ref, sh_ref, o_ref):
        o_ref[...] = jnp.maximum(h_ref[...] * sc_ref[...] + sh_ref[...], 0.0)

    return pl.pallas_call(
        body,
        grid=(npad // _TM,),
        in_specs=[
            pl.BlockSpec((_TM, c), lambda i: (i, 0)),
            pl.BlockSpec((c,), lambda i: (0,)),
            pl.BlockSpec((c,), lambda i: (0,)),
        ],
        out_specs=pl.BlockSpec((_TM, c), lambda i: (i, 0)),
        out_shape=jax.ShapeDtypeStruct((npad, c), jnp.float32),
    )(h, scale, shift)


def _tc_edge_gram(ea):
    e, c = ea.shape
    tm = 2048
    epad = _ceil_to(e, tm)
    if epad != e:
        ea = jnp.concatenate([ea, jnp.zeros((epad - e, c), ea.dtype)])
    grid = epad // tm

    def body(a_ref, g_ref, s_ref, acc):
        i = pl.program_id(0)
        a = a_ref[...]

        @pl.when(i == 0)
        def _():
            acc[...] = jnp.zeros_like(acc)

        acc[:c, :] += jax.lax.dot_general(
            a, a, (((0,), (0,)), ((), ())),
            preferred_element_type=jnp.float32)
        acc[c, :] += jnp.sum(a, 0)

        @pl.when(i == grid - 1)
        def _():
            g_ref[...] = acc[:c, :]
            s_ref[...] = acc[c, :]

    return pl.pallas_call(
        body,
        grid=(grid,),
        in_specs=[pl.BlockSpec((tm, c), lambda i: (i, 0))],
        out_specs=[
            pl.BlockSpec((c, c), lambda i: (0, 0)),
            pl.BlockSpec((c,), lambda i: (0,)),
        ],
        out_shape=[
            jax.ShapeDtypeStruct((c, c), jnp.float32),
            jax.ShapeDtypeStruct((c,), jnp.float32),
        ],
        scratch_shapes=[pltpu.VMEM((c + 1, c), jnp.float32)],
    )(ea)


def _tc_relation_fwd(parts, cnt16, xd, eadw, wl, bl, wr, etw, etb,
                     attw, attb, n_valid):
    ndp = xd.shape[0]
    c = xd.shape[1]
    cw = parts[0].shape[-1]
    nparts = len(parts)
    grid = ndp // _TM
    out_dim = wl.shape[1]

    def body(*refs):
        part_refs = refs[:nparts]
        (cnt_ref, xd_ref, eadw_ref, wl_ref, bl_ref, wr_ref, etw_ref,
         etb_ref, attw_ref, attb_ref, out_ref, st_ref, acc) = refs[nparts:]
        i = pl.program_id(0)
        if nparts > 1:
            s = jnp.concatenate([p[0] + p[1] for p in part_refs], axis=-1)
        else:
            s = part_refs[0][0] + part_refs[0][1]
        cnt = cnt_ref[0, :, 0:1] + cnt_ref[1, :, 0:1]
        mean = s / jnp.maximum(cnt, 1.0)
        out = (jnp.dot(mean, wl_ref[...], preferred_element_type=jnp.float32)
               + bl_ref[...]
               + jnp.dot(xd_ref[...], wr_ref[...],
                         preferred_element_type=jnp.float32))
        eat = jnp.dot(eadw_ref[...], etw_ref[...],
                      preferred_element_type=jnp.float32) + etb_ref[...]
        score = (jnp.dot(out, attw_ref[...][:out_dim, :],
                         preferred_element_type=jnp.float32)
                 + jnp.dot(eat, attw_ref[...][out_dim:, :],
                           preferred_element_type=jnp.float32)
                 + attb_ref[0])
        attn = jax.nn.sigmoid(score)
        out = out + jnp.where(cnt > 0.0, attn * eat, 0.0)
        out_ref[...] = out
        rows = jax.lax.broadcasted_iota(jnp.int32, (_TM, 1), 0) + i * _TM
        om = jnp.where(rows < n_valid, out, 0.0)

        @pl.when(i == 0)
        def _():
            acc[...] = jnp.zeros_like(acc)

        acc[0, :] += jnp.sum(om, 0)
        acc[1, :] += jnp.sum(om * om, 0)

        @pl.when(i == grid - 1)
        def _():
            st_ref[...] = acc[...]

    in_specs = (
        [pl.BlockSpec((2, _TM, cw), lambda i: (0, i, 0))] * nparts + [
            pl.BlockSpec((2, _TM, 16), lambda i: (0, i, 0)),
            pl.BlockSpec((_TM, c), lambda i: (i, 0)),
            pl.BlockSpec((_TM, 32), lambda i: (i, 0)),
            pl.BlockSpec((c, out_dim), lambda i: (0, 0)),
            pl.BlockSpec((out_dim,), lambda i: (0,)),
            pl.BlockSpec((c, out_dim), lambda i: (0, 0)),
            pl.BlockSpec((32, out_dim), lambda i: (0, 0)),
            pl.BlockSpec((out_dim,), lambda i: (0,)),
            pl.BlockSpec((2 * out_dim, 1), lambda i: (0, 0)),
            pl.BlockSpec((1,), lambda i: (0,)),
        ])
    return pl.pallas_call(
        body,
        grid=(grid,),
        in_specs=in_specs,
        out_specs=[
            pl.BlockSpec((_TM, out_dim), lambda i: (i, 0)),
            pl.BlockSpec((2, out_dim), lambda i: (0, 0)),
        ],
        out_shape=[
            jax.ShapeDtypeStruct((ndp, out_dim), jnp.float32),
            jax.ShapeDtypeStruct((2, out_dim), jnp.float32),
        ],
        scratch_shapes=[pltpu.VMEM((2, out_dim), jnp.float32)],
    )(*parts, cnt16, xd, eadw, wl, bl, wr, etw, etb, attw, attb)


def _tc_finalize(outs, scales, shifts, x_t, wself, bself, head=None):
    ndp, c = x_t.shape
    nrel = len(outs)
    out_dim = wself.shape[1]
    grid = ndp // _TM

    def body(*refs):
        o_refs = refs[:nrel]
        sc_refs = refs[nrel:2 * nrel]
        sh_refs = refs[2 * nrel:3 * nrel]
        rest = refs[3 * nrel:]
        if head is None:
            x_ref, ws_ref, bs_ref, out_ref = rest
        else:
            x_ref, ws_ref, bs_ref, hw_ref, hb_ref, ha_ref, out_ref = rest
        acc = (jnp.dot(x_ref[...], ws_ref[...],
                       preferred_element_type=jnp.float32) + bs_ref[...])
        for r in range(nrel):
            acc = acc + jnp.maximum(
                o_refs[r][...] * sc_refs[r][...] + sh_refs[r][...], 0.0)
        xn = jnp.maximum(acc, 0.0)
        if head is None:
            out_ref[...] = xn
        else:
            y = jnp.sum(xn * hw_ref[...], axis=1) + hb_ref[0]
            out_ref[...] = jnp.where(y >= 0.0, y, ha_ref[0] * y)

    in_specs = ([pl.BlockSpec((_TM, out_dim), lambda i: (i, 0))] * nrel
                + [pl.BlockSpec((out_dim,), lambda i: (0,))] * (2 * nrel)
                + [pl.BlockSpec((_TM, c), lambda i: (i, 0)),
                   pl.BlockSpec((c, out_dim), lambda i: (0, 0)),
                   pl.BlockSpec((out_dim,), lambda i: (0,))])
    args = list(outs) + list(scales) + list(shifts) + [x_t, wself, bself]
    if head is None:
        out_spec = pl.BlockSpec((_TM, out_dim), lambda i: (i, 0))
        out_shape = jax.ShapeDtypeStruct((ndp, out_dim), jnp.float32)
    else:
        hw, hb, ha = head
        in_specs += [pl.BlockSpec((out_dim,), lambda i: (0,)),
                     pl.BlockSpec((1,), lambda i: (0,)),
                     pl.BlockSpec((1,), lambda i: (0,))]
        args += [hw, hb, ha]
        out_spec = pl.BlockSpec((_TM,), lambda i: (i,))
        out_shape = jax.ShapeDtypeStruct((ndp,), jnp.float32)
    return pl.pallas_call(
        body,
        grid=(grid,),
        in_specs=in_specs,
        out_specs=out_spec,
        out_shape=out_shape,
    )(*args)


def _bn_scale_shift(stats, n, g, be, eps=1e-5):
    mu = stats[0] / n
    var = stats[1] / n - mu * mu
    scale = g / jnp.sqrt(var + eps)
    return scale, be - mu * scale

_NODE_TYPES = ('pfas_sites', 'sw_stations', 'gw_wells')
_EDGE_TYPES = (
    ('pfas_sites', 'gw_wells'),
    ('pfas_sites', 'sw_stations'),
    ('sw_stations', 'pfas_sites'),
    ('sw_stations', 'gw_wells'),
    ('gw_wells', 'sw_stations'),
    ('gw_wells', 'gw_wells'),
    ('gw_wells', 'pfas_sites'),
)


def _ek(e):
    return e[0] + '->' + e[1]


def _pad_rows(a, n):
    if a.shape[0] == n:
        return a
    return jnp.concatenate(
        [a, jnp.zeros((n - a.shape[0],) + a.shape[1:], a.dtype)])


def kernel(x_dict, edge_index, edge_attr, params):
    n_nodes = {t: x_dict[t].shape[0] for t in _NODE_TYPES}
    ndp = {t: _ceil_to(n_nodes[t] + 1, _ZR * _NS) for t in _NODE_TYPES}

    xd = {}
    for t in _NODE_TYPES:
        q = params['node_red'][t]
        xp = _pad_rows(x_dict[t], ndp[t])
        h, st = _tc_linear_stats(xp, q['w'], q['b'], n_nodes[t])
        sc, sh = _bn_scale_shift(st, n_nodes[t], q['g'], q['be'])
        xd[t] = _tc_scale_shift_act(h, sc, sh)

    cnt16 = {}
    ead_win = {}
    rowp = {}
    colp = {}
    for e in _EDGE_TYPES:
        k = _ek(e)
        ei = edge_index[k]
        nd = n_nodes[e[1]]
        ndpd = ndp[e[1]]
        E = ei.shape[1]
        e_pad = _ceil_to(E, _NW * _K)
        rowp[k] = _pad1(ei[0], e_pad, 0).reshape(e_pad // _K, _K)
        colp[k] = _pad1(ei[1], e_pad, ndpd - 1).reshape(e_pad // _K, _K)
        cnt16[k] = _count_kernel(ndpd, e_pad)(colp[k])
        wtabs = _winner_kernel(ndpd, e_pad)(colp[k].reshape(-1))
        n_idx = _ceil_to(ndpd, _NW * _K)
        wsafe = _pad1(_tc_winner_combine(wtabs, E - 1), n_idx, 0)
        ea = edge_attr[k]
        ea_w = _gather16_kernel(E, n_idx)(
            ea, wsafe.reshape(n_idx // _K, _K))[:ndpd]
        q = params['edge_red'][k]
        G, su = _tc_edge_gram(ea)
        mu_ea = su / E
        cov = G / E - mu_ea[:, None] * mu_ea[None, :]
        mean_h = mu_ea @ q['w'] + q['b']
        var_h = jnp.sum(q['w'] * (cov @ q['w']), 0)
        sc_e = q['g'] / jnp.sqrt(var_h + 1e-5)
        sh_e = q['be'] - mean_h * sc_e
        h_w, _ = _tc_linear_stats(ea_w, q['w'], q['b'], ndp[e[1]])
        ead_win[k] = _tc_scale_shift_act(h_w, sc_e, sh_e)

    x = xd
    heads = {}
    for layer in ('conv1', 'conv2'):
        pl_ = params[layer]
        rel_out = {t: [] for t in _NODE_TYPES}
        rel_sc = {t: [] for t in _NODE_TYPES}
        rel_sh = {t: [] for t in _NODE_TYPES}
        for e in _EDGE_TYPES:
            k = _ek(e)
            p = pl_[k]
            dst = e[1]
            nd = n_nodes[dst]
            parts = _sc_segsum_parts(x[e[0]], rowp[k], colp[k], ndp[dst])
            o, st = _tc_relation_fwd(
                parts, cnt16[k], x[dst], ead_win[k],
                p['lin_l_w'], p['lin_l_b'], p['lin_r_w'],
                p['et_w'], p['et_b'], p['att_w'], p['att_b'], nd)
            sc, sh = _bn_scale_shift(st, nd, p['bn_g'], p['bn_b'])
            rel_out[dst].append(o)
            rel_sc[dst].append(2.0 * sc)
            rel_sh[dst].append(2.0 * sh)
        xn = {}
        for t in _NODE_TYPES:
            p = pl_['self:' + t]
            head = None
            if layer == 'conv2' and t != 'pfas_sites':
                head = (params['linear']['w'][:, 0],
                        params['linear']['b'],
                        params['prelu'].reshape((1,)))
            res = _tc_finalize(rel_out[t], rel_sc[t], rel_sh[t], x[t],
                               p['lin_l_w'] + p['lin_r_w'], p['lin_l_b'],
                               head=head)
            if head is None:
                xn[t] = res
            else:
                heads[t] = res
        x = xn

    pfas = x['pfas_sites'][:n_nodes['pfas_sites']]
    sw = heads['sw_stations'][:n_nodes['sw_stations'], None]
    gw = heads['gw_wells'][:n_nodes['gw_wells'], None]
    return pfas, sw, gw


# speedup vs baseline: 1.0663x; 1.0322x over previous
import dataclasses
import functools

import jax
import jax.numpy as jnp
from jax import lax
from jax.experimental import pallas as pl
from jax.experimental.pallas import tpu as pltpu
from jax.experimental.pallas import tpu_sc as plsc

_NC, _NS, _L = 2, 16, 16
_NW = _NC * _NS
_K = 128
_ZR = 64
_SPMEM_WORDS = 2_075_000


def _sc_mesh():
    return plsc.VectorSubcoreMesh(core_axis_name="c", subcore_axis_name="s")


def _sc_params(layout_passes=True):
    cp = pltpu.CompilerParams(use_tc_tiling_on_sc=False)
    if not layout_passes and (
            "needs_layout_passes" in pltpu.CompilerParams.__dataclass_fields__):
        cp = dataclasses.replace(cp, needs_layout_passes=False)
    return cp


@functools.cache
def _segsum_kernel(c, nd_pad, e_pad, nb=2):
    epw = e_pad // _NW
    nch = epw // _K
    rps = nd_pad // _NS

    @functools.partial(
        pl.kernel,
        out_type=jax.ShapeDtypeStruct((_NC, nd_pad, c), jnp.float32),
        mesh=_sc_mesh(),
        scratch_types=[
            pltpu.VMEM((nch, _K), jnp.int32),
            pltpu.VMEM((nch, _K), jnp.int32),
        ] + [pltpu.VMEM((_K, c), jnp.float32)] * nb + [
            pltpu.VMEM((_ZR, c), jnp.float32),
            pltpu.VMEM_SHARED((nd_pad, c), jnp.float32),
        ] + [pltpu.SemaphoreType.DMA] * (2 * nb),
        compiler_params=_sc_params(),
    )
    def k(x_hbm, row_hbm, col_hbm, out_hbm, row_v, col_v, *rest):
        bufs = rest[:nb]
        ztile = rest[nb]
        acc = rest[nb + 1]
        gsems = rest[nb + 2:2 * nb + 2]
        ssems = rest[2 * nb + 2:]
        cid = lax.axis_index("c")
        sid = lax.axis_index("s")
        zv = jnp.zeros((_L,), jnp.float32)

        @pl.loop(0, _ZR)
        def _(i):
            @pl.loop(0, c, step=_L)
            def _(j):
                ztile[i, pl.ds(j, _L)] = zv

        rbase = sid * rps

        @pl.loop(0, rps, step=_ZR)
        def _(r):
            pltpu.sync_copy(ztile, acc.at[pl.ds(rbase + r, _ZR)])

        wid = sid * _NC + cid
        pltpu.sync_copy(row_hbm.at[pl.ds(wid * nch, nch)], row_v)
        pltpu.sync_copy(col_hbm.at[pl.ds(wid * nch, nch)], col_v)
        plsc.subcore_barrier()

        hg = [None] * nch
        hs = [None] * nch
        for i in range(min(nb, nch)):
            hg[i] = pltpu.async_copy(x_hbm.at[row_v.at[i]], bufs[i],
                                     gsems[i])
        for i in range(nch):
            b = i % nb
            hg[i].wait()
            if i >= 1:
                hs[i - 1].wait()
                f = i + nb - 1
                if f < nch:
                    fb = f % nb
                    hg[f] = pltpu.async_copy(x_hbm.at[row_v.at[f]],
                                             bufs[fb], gsems[fb])
            hs[i] = pltpu.async_copy(bufs[b], acc.at[col_v.at[i]],
                                     ssems[b], add=True)
        hs[nch - 1].wait()
        plsc.subcore_barrier()
        pltpu.sync_copy(acc.at[pl.ds(rbase, rps)],
                        out_hbm.at[cid].at[pl.ds(rbase, rps)])

    return k


def _pad1(a, n, fill):
    if n == a.shape[0]:
        return a
    return jnp.concatenate(
        [a, jnp.full((n - a.shape[0],), fill, a.dtype)])


def _ceil_to(x, m):
    return -(-x // m) * m


def _sc_segsum_parts(x, rowp, colp, nd_pad):
    ns, c = x.shape
    e_pad = rowp.shape[0] * rowp.shape[1]
    nch = e_pad // _NW // _K

    def words(cw, nb):
        return _NS * (2 * nch * _K + nb * _K * cw + _ZR * cw) + nd_pad * cw

    cw = c
    while words(cw, 2) > _SPMEM_WORDS:
        cw //= 2
    return [_segsum_kernel(cw, nd_pad, e_pad, 2)(x[:, i:i + cw], rowp, colp)
            for i in range(0, c, cw)]


@functools.cache
def _count_kernel(nd_pad, e_pad):
    epw = e_pad // _NW
    nchunks = epw // _K
    rps = nd_pad // _NS

    @functools.partial(
        pl.kernel,
        out_type=jax.ShapeDtypeStruct((_NC, nd_pad, 16), jnp.float32),
        mesh=_sc_mesh(),
        scratch_types=[
            pltpu.VMEM((nchunks, _K), jnp.int32),
            pltpu.VMEM((_K, 16), jnp.float32),
            pltpu.VMEM((_ZR, 16), jnp.float32),
            pltpu.VMEM_SHARED((nd_pad, 16), jnp.float32),
            pltpu.SemaphoreType.DMA,
        ],
        compiler_params=pltpu.CompilerParams(use_tc_tiling_on_sc=False),
    )
    def k(col_hbm, out_hbm, col_v, ones_v, ztile, acc, sem):
        cid = lax.axis_index("c")
        sid = lax.axis_index("s")
        zv = jnp.zeros((_L,), jnp.float32)
        ov = jnp.ones((_L,), jnp.float32)

        @pl.loop(0, _ZR)
        def _(i):
            ztile[i, pl.ds(0, _L)] = zv

        @pl.loop(0, _K)
        def _(i):
            ones_v[i, pl.ds(0, _L)] = ov

        rbase = sid * rps

        @pl.loop(0, rps, step=_ZR)
        def _(r):
            pltpu.sync_copy(ztile, acc.at[pl.ds(rbase + r, _ZR)])

        wid = sid * _NC + cid
        pltpu.sync_copy(col_hbm.at[pl.ds(wid * nchunks, nchunks)], col_v)
        plsc.subcore_barrier()

        hs = [pltpu.async_copy(ones_v, acc.at[col_v.at[i]], sem, add=True)
              for i in range(nchunks)]
        for h in hs:
            h.wait()

        plsc.subcore_barrier()
        pltpu.sync_copy(acc.at[pl.ds(rbase, rps)],
                        out_hbm.at[cid].at[pl.ds(rbase, rps)])

    return k


@functools.cache
def _winner_kernel(nd_pad, e_pad):
    epw = e_pad // _NW
    nchunks = epw // _K

    @functools.partial(
        pl.kernel,
        out_type=jax.ShapeDtypeStruct((_NW, nd_pad), jnp.int32),
        mesh=_sc_mesh(),
        scratch_types=[
            pltpu.VMEM((epw,), jnp.int32),
            pltpu.VMEM((nd_pad,), jnp.int32),
        ],
        compiler_params=_sc_params(layout_passes=False),
    )
    def k(col_hbm, out_hbm, col_v, wtab):
        cid = lax.axis_index("c")
        sid = lax.axis_index("s")
        wid = sid * _NC + cid
        neg = jnp.full((_L,), -1, jnp.int32)
        pltpu.sync_copy(col_hbm.at[pl.ds(wid * epw, epw)], col_v)

        @pl.loop(0, nd_pad, step=_L)
        def _(i):
            wtab[pl.ds(i, _L)] = neg

        iota = lax.iota(jnp.int32, _L)
        nxt_idx = jnp.minimum(iota + 1, _L - 1)
        base = wid * epw

        @pl.loop(0, epw, step=_L)
        def _(j):
            if True:
                c = col_v[pl.ds(j, _L)]
                eid = base + j + iota
                key = (c << 4) | iota
                sk, se = plsc.sort_key_val(key, eid)
                cs = sk >> 4
                nxt = lax.gather(
                    cs, nxt_idx[:, None],
                    lax.GatherDimensionNumbers(
                        offset_dims=(), collapsed_slice_dims=(0,),
                        start_index_map=(0,)),
                    slice_sizes=(1,),
                    mode=lax.GatherScatterMode.PROMISE_IN_BOUNDS)
                lastm = (cs != nxt) | (iota == _L - 1)
                plsc.store_scatter(wtab, [cs], se, mask=lastm)

        pltpu.sync_copy(wtab, out_hbm.at[wid])

    return k


@functools.cache
def _gather16_kernel(ne, n_idx):
    ipw = n_idx // _NW
    nchunks = ipw // _K

    @functools.partial(
        pl.kernel,
        out_type=jax.ShapeDtypeStruct((n_idx, 16), jnp.float32),
        mesh=_sc_mesh(),
        scratch_types=[
            pltpu.VMEM((nchunks, _K), jnp.int32),
            pltpu.VMEM((_K, 16), jnp.float32),
            pltpu.VMEM((_K, 16), jnp.float32),
            pltpu.SemaphoreType.DMA,
            pltpu.SemaphoreType.DMA,
            pltpu.SemaphoreType.DMA,
            pltpu.SemaphoreType.DMA,
        ],
        compiler_params=pltpu.CompilerParams(use_tc_tiling_on_sc=False),
    )
    def k(tab_hbm, idx_hbm, out_hbm, idx_v, ga, gb, gsa, gsb, osa, osb):
        cid = lax.axis_index("c")
        sid = lax.axis_index("s")
        wid = sid * _NC + cid
        base = wid * ipw
        pltpu.sync_copy(idx_hbm.at[pl.ds(wid * nchunks, nchunks)], idx_v)

        bufs = (ga, gb)
        gsems = (gsa, gsb)
        osems = (osa, osb)
        hg = [None] * nchunks
        ho = [None] * nchunks
        hg[0] = pltpu.async_copy(tab_hbm.at[idx_v.at[0]], ga, gsa)
        for i in range(nchunks):
            b = i % 2
            hg[i].wait()
            if i >= 1:
                ho[i - 1].wait()
            if i + 1 < nchunks:
                nb = (i + 1) % 2
                hg[i + 1] = pltpu.async_copy(
                    tab_hbm.at[idx_v.at[i + 1]], bufs[nb], gsems[nb])
            ho[i] = pltpu.async_copy(
                bufs[b], out_hbm.at[pl.ds(base + i * _K, _K)], osems[b])
        ho[nchunks - 1].wait()

    return k


def _tc_winner_combine(wtabs, emax):
    nw, ndp = wtabs.shape
    tmc = 1024

    def body(w_ref, o_ref):
        o_ref[...] = jnp.clip(jnp.max(w_ref[...], axis=0), 0, emax)

    return pl.pallas_call(
        body,
        grid=(ndp // tmc,),
        in_specs=[pl.BlockSpec((nw, tmc), lambda i: (0, i))],
        out_specs=pl.BlockSpec((tmc,), lambda i: (i,)),
        out_shape=jax.ShapeDtypeStruct((ndp,), jnp.int32),
    )(wtabs)

_TM = 512


def _tc_linear_stats(x, w, b, n_valid):
    npad, cin = x.shape
    cout = w.shape[1]
    grid = npad // _TM

    def body(x_ref, w_ref, b_ref, h_ref, st_ref, acc):
        i = pl.program_id(0)
        h = jnp.dot(x_ref[...], w_ref[...],
                    preferred_element_type=jnp.float32) + b_ref[...]
        h_ref[...] = h
        rows = jax.lax.broadcasted_iota(jnp.int32, (_TM, 1), 0) + i * _TM
        hm = jnp.where(rows < n_valid, h, 0.0)

        @pl.when(i == 0)
        def _():
            acc[...] = jnp.zeros_like(acc)

        acc[0, :] += jnp.sum(hm, 0)
        acc[1, :] += jnp.sum(hm * hm, 0)

        @pl.when(i == grid - 1)
        def _():
            st_ref[...] = acc[...]

    return pl.pallas_call(
        body,
        grid=(grid,),
        in_specs=[
            pl.BlockSpec((_TM, cin), lambda i: (i, 0)),
            pl.BlockSpec((cin, cout), lambda i: (0, 0)),
            pl.BlockSpec((cout,), lambda i: (0,)),
        ],
        out_specs=[
            pl.BlockSpec((_TM, cout), lambda i: (i, 0)),
            pl.BlockSpec((2, cout), lambda i: (0, 0)),
        ],
        out_shape=[
            jax.ShapeDtypeStruct((npad, cout), jnp.float32),
            jax.ShapeDtypeStruct((2, cout), jnp.float32),
        ],
        scratch_shapes=[pltpu.VMEM((2, cout), jnp.float32)],
    )(x, w, b)


def _tc_scale_shift_act(h, scale, shift):
    npad, c = h.shape

    def body(h_ref, sc_ref, sh_ref, o_ref):
        o_ref[...] = jnp.maximum(h_ref[...] * sc_ref[...] + sh_ref[...], 0.0)

    return pl.pallas_call(
        body,
        grid=(npad // _TM,),
        in_specs=[
            pl.BlockSpec((_TM, c), lambda i: (i, 0)),
            pl.BlockSpec((c,), lambda i: (0,)),
            pl.BlockSpec((c,), lambda i: (0,)),
        ],
        out_specs=pl.BlockSpec((_TM, c), lambda i: (i, 0)),
        out_shape=jax.ShapeDtypeStruct((npad, c), jnp.float32),
    )(h, scale, shift)


def _tc_edge_gram(ea):
    e, c = ea.shape
    tm = 2048
    epad = _ceil_to(e, tm)
    if epad != e:
        ea = jnp.concatenate([ea, jnp.zeros((epad - e, c), ea.dtype)])
    grid = epad // tm

    def body(a_ref, g_ref, s_ref, acc):
        i = pl.program_id(0)
        a = a_ref[...]

        @pl.when(i == 0)
        def _():
            acc[...] = jnp.zeros_like(acc)

        acc[:c, :] += jax.lax.dot_general(
            a, a, (((0,), (0,)), ((), ())),
            preferred_element_type=jnp.float32)
        acc[c, :] += jnp.sum(a, 0)

        @pl.when(i == grid - 1)
        def _():
            g_ref[...] = acc[:c, :]
            s_ref[...] = acc[c, :]

    return pl.pallas_call(
        body,
        grid=(grid,),
        in_specs=[pl.BlockSpec((tm, c), lambda i: (i, 0))],
        out_specs=[
            pl.BlockSpec((c, c), lambda i: (0, 0)),
            pl.BlockSpec((c,), lambda i: (0,)),
        ],
        out_shape=[
            jax.ShapeDtypeStruct((c, c), jnp.float32),
            jax.ShapeDtypeStruct((c,), jnp.float32),
        ],
        scratch_shapes=[pltpu.VMEM((c + 1, c), jnp.float32)],
    )(ea)


def _tc_relation_fwd(parts, cnt16, xd, eadw, wl, bl, wr, etw, etb,
                     attw, attb, n_valid):
    ndp = xd.shape[0]
    c = xd.shape[1]
    cw = parts[0].shape[-1]
    nparts = len(parts)
    grid = ndp // _TM
    out_dim = wl.shape[1]

    def body(*refs):
        part_refs = refs[:nparts]
        (cnt_ref, xd_ref, eadw_ref, wl_ref, bl_ref, wr_ref, etw_ref,
         etb_ref, attw_ref, attb_ref, out_ref, st_ref, acc) = refs[nparts:]
        i = pl.program_id(0)
        if nparts > 1:
            s = jnp.concatenate([p[0] + p[1] for p in part_refs], axis=-1)
        else:
            s = part_refs[0][0] + part_refs[0][1]
        cnt = cnt_ref[0, :, 0:1] + cnt_ref[1, :, 0:1]
        mean = s / jnp.maximum(cnt, 1.0)
        out = (jnp.dot(mean, wl_ref[...], preferred_element_type=jnp.float32)
               + bl_ref[...]
               + jnp.dot(xd_ref[...], wr_ref[...],
                         preferred_element_type=jnp.float32))
        eat = jnp.dot(eadw_ref[...], etw_ref[...],
                      preferred_element_type=jnp.float32) + etb_ref[...]
        score = (jnp.dot(out, attw_ref[...][:out_dim, :],
                         preferred_element_type=jnp.float32)
                 + jnp.dot(eat, attw_ref[...][out_dim:, :],
                           preferred_element_type=jnp.float32)
                 + attb_ref[0])
        attn = jax.nn.sigmoid(score)
        out = out + jnp.where(cnt > 0.0, attn * eat, 0.0)
        out_ref[...] = out
        rows = jax.lax.broadcasted_iota(jnp.int32, (_TM, 1), 0) + i * _TM
        om = jnp.where(rows < n_valid, out, 0.0)

        @pl.when(i == 0)
        def _():
            acc[...] = jnp.zeros_like(acc)

        acc[0, :] += jnp.sum(om, 0)
        acc[1, :] += jnp.sum(om * om, 0)

        @pl.when(i == grid - 1)
        def _():
            st_ref[...] = acc[...]

    in_specs = (
        [pl.BlockSpec((2, _TM, cw), lambda i: (0, i, 0))] * nparts + [
            pl.BlockSpec((2, _TM, 16), lambda i: (0, i, 0)),
            pl.BlockSpec((_TM, c), lambda i: (i, 0)),
            pl.BlockSpec((_TM, 32), lambda i: (i, 0)),
            pl.BlockSpec((c, out_dim), lambda i: (0, 0)),
            pl.BlockSpec((out_dim,), lambda i: (0,)),
            pl.BlockSpec((c, out_dim), lambda i: (0, 0)),
            pl.BlockSpec((32, out_dim), lambda i: (0, 0)),
            pl.BlockSpec((out_dim,), lambda i: (0,)),
            pl.BlockSpec((2 * out_dim, 1), lambda i: (0, 0)),
            pl.BlockSpec((1,), lambda i: (0,)),
        ])
    return pl.pallas_call(
        body,
        grid=(grid,),
        in_specs=in_specs,
        out_specs=[
            pl.BlockSpec((_TM, out_dim), lambda i: (i, 0)),
            pl.BlockSpec((2, out_dim), lambda i: (0, 0)),
        ],
        out_shape=[
            jax.ShapeDtypeStruct((ndp, out_dim), jnp.float32),
            jax.ShapeDtypeStruct((2, out_dim), jnp.float32),
        ],
        scratch_shapes=[pltpu.VMEM((2, out_dim), jnp.float32)],
    )(*parts, cnt16, xd, eadw, wl, bl, wr, etw, etb, attw, attb)


def _tc_finalize(outs, scales, shifts, x_t, wself, bself, head=None):
    ndp, c = x_t.shape
    nrel = len(outs)
    out_dim = wself.shape[1]
    grid = ndp // _TM

    def body(*refs):
        o_refs = refs[:nrel]
        sc_refs = refs[nrel:2 * nrel]
        sh_refs = refs[2 * nrel:3 * nrel]
        rest = refs[3 * nrel:]
        if head is None:
            x_ref, ws_ref, bs_ref, out_ref = rest
        else:
            x_ref, ws_ref, bs_ref, hw_ref, hb_ref, ha_ref, out_ref = rest
        acc = (jnp.dot(x_ref[...], ws_ref[...],
                       preferred_element_type=jnp.float32) + bs_ref[...])
        for r in range(nrel):
            acc = acc + jnp.maximum(
                o_refs[r][...] * sc_refs[r][...] + sh_refs[r][...], 0.0)
        xn = jnp.maximum(acc, 0.0)
        if head is None:
            out_ref[...] = xn
        else:
            y = jnp.sum(xn * hw_ref[...], axis=1) + hb_ref[0]
            out_ref[...] = jnp.where(y >= 0.0, y, ha_ref[0] * y)

    in_specs = ([pl.BlockSpec((_TM, out_dim), lambda i: (i, 0))] * nrel
                + [pl.BlockSpec((out_dim,), lambda i: (0,))] * (2 * nrel)
                + [pl.BlockSpec((_TM, c), lambda i: (i, 0)),
                   pl.BlockSpec((c, out_dim), lambda i: (0, 0)),
                   pl.BlockSpec((out_dim,), lambda i: (0,))])
    args = list(outs) + list(scales) + list(shifts) + [x_t, wself, bself]
    if head is None:
        out_spec = pl.BlockSpec((_TM, out_dim), lambda i: (i, 0))
        out_shape = jax.ShapeDtypeStruct((ndp, out_dim), jnp.float32)
    else:
        hw, hb, ha = head
        in_specs += [pl.BlockSpec((out_dim,), lambda i: (0,)),
                     pl.BlockSpec((1,), lambda i: (0,)),
                     pl.BlockSpec((1,), lambda i: (0,))]
        args += [hw, hb, ha]
        out_spec = pl.BlockSpec((_TM,), lambda i: (i,))
        out_shape = jax.ShapeDtypeStruct((ndp,), jnp.float32)
    return pl.pallas_call(
        body,
        grid=(grid,),
        in_specs=in_specs,
        out_specs=out_spec,
        out_shape=out_shape,
    )(*args)


def _bn_scale_shift(stats, n, g, be, eps=1e-5):
    mu = stats[0] / n
    var = stats[1] / n - mu * mu
    scale = g / jnp.sqrt(var + eps)
    return scale, be - mu * scale

_NODE_TYPES = ('pfas_sites', 'sw_stations', 'gw_wells')
_EDGE_TYPES = (
    ('pfas_sites', 'gw_wells'),
    ('pfas_sites', 'sw_stations'),
    ('sw_stations', 'pfas_sites'),
    ('sw_stations', 'gw_wells'),
    ('gw_wells', 'sw_stations'),
    ('gw_wells', 'gw_wells'),
    ('gw_wells', 'pfas_sites'),
)


def _ek(e):
    return e[0] + '->' + e[1]


def _pad_rows(a, n):
    if a.shape[0] == n:
        return a
    return jnp.concatenate(
        [a, jnp.zeros((n - a.shape[0],) + a.shape[1:], a.dtype)])


def kernel(x_dict, edge_index, edge_attr, params):
    n_nodes = {t: x_dict[t].shape[0] for t in _NODE_TYPES}
    ndp = {t: _ceil_to(n_nodes[t] + 1, _ZR * _NS) for t in _NODE_TYPES}

    xd = {}
    for t in _NODE_TYPES:
        q = params['node_red'][t]
        xp = _pad_rows(x_dict[t], ndp[t])
        h, st = _tc_linear_stats(xp, q['w'], q['b'], n_nodes[t])
        sc, sh = _bn_scale_shift(st, n_nodes[t], q['g'], q['be'])
        xd[t] = _tc_scale_shift_act(h, sc, sh)

    cnt16 = {}
    ead_win = {}
    rowp = {}
    colp = {}
    for e in _EDGE_TYPES:
        k = _ek(e)
        ei = edge_index[k]
        nd = n_nodes[e[1]]
        ndpd = ndp[e[1]]
        E = ei.shape[1]
        e_pad = _ceil_to(E, _NW * _K)
        rowp[k] = _pad1(ei[0], e_pad, 0).reshape(e_pad // _K, _K)
        colp[k] = _pad1(ei[1], e_pad, ndpd - 1).reshape(e_pad // _K, _K)
        cnt16[k] = _count_kernel(ndpd, e_pad)(colp[k])
        wtabs = _winner_kernel(ndpd, e_pad)(colp[k].reshape(-1))
        n_idx = _ceil_to(ndpd, _NW * _K)
        wsafe = _pad1(_tc_winner_combine(wtabs, E - 1), n_idx, 0)
        ea = edge_attr[k]
        ea_w = _gather16_kernel(E, n_idx)(
            ea, wsafe.reshape(n_idx // _K, _K))[:ndpd]
        q = params['edge_red'][k]
        G, su = _tc_edge_gram(ea)
        mu_ea = su / E
        cov = G / E - mu_ea[:, None] * mu_ea[None, :]
        mean_h = mu_ea @ q['w'] + q['b']
        var_h = jnp.sum(q['w'] * (cov @ q['w']), 0)
        sc_e = q['g'] / jnp.sqrt(var_h + 1e-5)
        sh_e = q['be'] - mean_h * sc_e
        h_w, _ = _tc_linear_stats(ea_w, q['w'], q['b'], ndp[e[1]])
        ead_win[k] = _tc_scale_shift_act(h_w, sc_e, sh_e)

    x = xd
    heads = {}
    for layer in ('conv1', 'conv2'):
        pl_ = params[layer]
        rel_out = {t: [] for t in _NODE_TYPES}
        rel_sc = {t: [] for t in _NODE_TYPES}
        rel_sh = {t: [] for t in _NODE_TYPES}
        for e in _EDGE_TYPES:
            k = _ek(e)
            p = pl_[k]
            dst = e[1]
            nd = n_nodes[dst]
            parts = _sc_segsum_parts(x[e[0]], rowp[k], colp[k], ndp[dst])
            o, st = _tc_relation_fwd(
                parts, cnt16[k], x[dst], ead_win[k],
                p['lin_l_w'], p['lin_l_b'], p['lin_r_w'],
                p['et_w'], p['et_b'], p['att_w'], p['att_b'], nd)
            sc, sh = _bn_scale_shift(st, nd, p['bn_g'], p['bn_b'])
            rel_out[dst].append(o)
            rel_sc[dst].append(2.0 * sc)
            rel_sh[dst].append(2.0 * sh)
        xn = {}
        for t in _NODE_TYPES:
            p = pl_['self:' + t]
            head = None
            if layer == 'conv2' and t != 'pfas_sites':
                head = (params['linear']['w'][:, 0],
                        params['linear']['b'],
                        params['prelu'].reshape((1,)))
            res = _tc_finalize(rel_out[t], rel_sc[t], rel_sh[t], x[t],
                               p['lin_l_w'] + p['lin_r_w'], p['lin_l_b'],
                               head=head)
            if head is None:
                xn[t] = res
            else:
                heads[t] = res
        x = xn

    pfas = x['pfas_sites'][:n_nodes['pfas_sites']]
    sw = heads['sw_stations'][:n_nodes['sw_stations'], None]
    gw = heads['gw_wells'][:n_nodes['gw_wells'], None]
    return pfas, sw, gw
